# Initial kernel scaffold; baseline (speedup 1.0000x reference)
import jax
import jax.numpy as jnp
from jax.experimental import pallas as pl


def kernel(x, adj, W1, b1, W2, b2):
    raise NotImplementedError("write your pallas kernel here")


# speedup vs baseline: 2336.7123x; 2336.7123x over previous
import jax
import jax.numpy as jnp
from jax.experimental import pallas as pl

_DOT_T = (((0,), (0,)), ((), ()))


def _gcn_body(x_ref, adj_ref, w1_ref, b1_ref, w2_ref, b2_ref, out_ref):
    A = adj_ref[0]
    X = x_ref[0]
    deg = jnp.sum(A, axis=0) + 1.0
    dinv = jnp.where(deg > 0, jax.lax.rsqrt(deg), 0.0)[:, None]

    def layer(h, W, b):
        t = dinv * jnp.dot(h, W, preferred_element_type=jnp.float32)
        msg = jax.lax.dot_general(A, t, _DOT_T, preferred_element_type=jnp.float32)
        return dinv * (msg + t) + b

    h = jax.nn.relu(layer(X, w1_ref[...], b1_ref[...]))
    h = layer(h, w2_ref[...], b2_ref[...])
    out_ref[0] = jnp.mean(h, axis=0, keepdims=True)


def kernel(x, adj, W1, b1, W2, b2):
    B, N, F = x.shape
    O = W2.shape[1]
    return pl.pallas_call(
        _gcn_body,
        grid=(B,),
        in_specs=[
            pl.BlockSpec((1, N, F), lambda b: (b, 0, 0)),
            pl.BlockSpec((1, N, N), lambda b: (b, 0, 0)),
            pl.BlockSpec(W1.shape, lambda b: (0, 0)),
            pl.BlockSpec((1, b1.shape[0]), lambda b: (0, 0)),
            pl.BlockSpec(W2.shape, lambda b: (0, 0)),
            pl.BlockSpec((1, b2.shape[0]), lambda b: (0, 0)),
        ],
        out_specs=pl.BlockSpec((1, 1, O), lambda b: (b, 0, 0)),
        out_shape=jax.ShapeDtypeStruct((B, 1, O), x.dtype),
    )(x, adj, W1, b1.reshape(1, -1), W2, b2.reshape(1, -1)).reshape(B, O)


# speedup vs baseline: 3749.0583x; 1.6044x over previous
import jax
import jax.numpy as jnp
from jax.experimental import pallas as pl

_PER_PROG = 2
_DOT_MSG = (((1,), (1,)), ((0,), (0,)))
_DOT_XW = (((2,), (0,)), ((), ()))


def _gcn_body(x_ref, adj_ref, w1_ref, b1_ref, w2_ref, b2_ref, out_ref):
    A = adj_ref[...]
    X = x_ref[...]
    deg = jnp.sum(A, axis=1) + 1.0
    dinv = jnp.where(deg > 0, jax.lax.rsqrt(deg), 0.0)[..., None]

    def layer(h, W, b):
        t = dinv * jax.lax.dot_general(h, W, _DOT_XW,
                                       preferred_element_type=jnp.float32)
        msg = jax.lax.dot_general(A, t, _DOT_MSG,
                                  preferred_element_type=jnp.float32)
        return dinv * (msg + t) + b

    h = jax.nn.relu(layer(X, w1_ref[...], b1_ref[...]))
    h = layer(h, w2_ref[...], b2_ref[...])
    out_ref[0] = jnp.mean(h, axis=1)


def kernel(x, adj, W1, b1, W2, b2):
    B, N, F = x.shape
    O = W2.shape[1]
    g = _PER_PROG
    return pl.pallas_call(
        _gcn_body,
        grid=(B // g,),
        in_specs=[
            pl.BlockSpec((g, N, F), lambda b: (b, 0, 0)),
            pl.BlockSpec((g, N, N), lambda b: (b, 0, 0)),
            pl.BlockSpec(W1.shape, lambda b: (0, 0)),
            pl.BlockSpec((1, b1.shape[0]), lambda b: (0, 0)),
            pl.BlockSpec(W2.shape, lambda b: (0, 0)),
            pl.BlockSpec((1, b2.shape[0]), lambda b: (0, 0)),
        ],
        out_specs=pl.BlockSpec((1, g, O), lambda b: (b, 0, 0)),
        out_shape=jax.ShapeDtypeStruct((B // g, g, O), x.dtype),
    )(x, adj, W1, b1.reshape(1, -1), W2, b2.reshape(1, -1)).reshape(B, O)


# speedup vs baseline: 5609.2826x; 1.4962x over previous
import jax
import jax.numpy as jnp
from jax.experimental import pallas as pl

_PER_PROG = 4
_DOT_MSG = (((1,), (1,)), ((0,), (0,)))
_DOT_XW = (((2,), (0,)), ((), ()))


def _gcn_body(x_ref, adj_ref, w1_ref, b1_ref, w2_ref, b2_ref, out_ref):
    A = adj_ref[...]
    X = x_ref[...]
    deg = jnp.sum(A, axis=1) + 1.0
    dinv = jnp.where(deg > 0, jax.lax.rsqrt(deg), 0.0)[..., None]

    def layer(h, W, b):
        t = dinv * jax.lax.dot_general(h, W, _DOT_XW,
                                       preferred_element_type=jnp.float32)
        msg = jax.lax.dot_general(A, t, _DOT_MSG,
                                  preferred_element_type=jnp.float32)
        return dinv * (msg + t) + b

    h = jax.nn.relu(layer(X, w1_ref[...], b1_ref[...]))
    h = layer(h, w2_ref[...], b2_ref[...])
    out_ref[0] = jnp.mean(h, axis=1)


def kernel(x, adj, W1, b1, W2, b2):
    B, N, F = x.shape
    O = W2.shape[1]
    g = _PER_PROG
    return pl.pallas_call(
        _gcn_body,
        grid=(B // g,),
        in_specs=[
            pl.BlockSpec((g, N, F), lambda b: (b, 0, 0)),
            pl.BlockSpec((g, N, N), lambda b: (b, 0, 0)),
            pl.BlockSpec(W1.shape, lambda b: (0, 0)),
            pl.BlockSpec((1, b1.shape[0]), lambda b: (0, 0)),
            pl.BlockSpec(W2.shape, lambda b: (0, 0)),
            pl.BlockSpec((1, b2.shape[0]), lambda b: (0, 0)),
        ],
        out_specs=pl.BlockSpec((1, g, O), lambda b: (b, 0, 0)),
        out_shape=jax.ShapeDtypeStruct((B // g, g, O), x.dtype),
    )(x, adj, W1, b1.reshape(1, -1), W2, b2.reshape(1, -1)).reshape(B, O)


# speedup vs baseline: 6552.1826x; 1.1681x over previous
import jax
import jax.numpy as jnp
from jax.experimental import pallas as pl

_PER_PROG = 8
_DOT_MSG = (((1,), (1,)), ((0,), (0,)))
_DOT_XW = (((2,), (0,)), ((), ()))


def _gcn_body(x_ref, adj_ref, w1_ref, b1_ref, w2_ref, b2_ref, out_ref):
    A = adj_ref[...]
    X = x_ref[...]
    deg = jnp.sum(A, axis=1) + 1.0
    dinv = jnp.where(deg > 0, jax.lax.rsqrt(deg), 0.0)[..., None]

    def layer(h, W, b):
        t = dinv * jax.lax.dot_general(h, W, _DOT_XW,
                                       preferred_element_type=jnp.float32)
        msg = jax.lax.dot_general(A, t, _DOT_MSG,
                                  preferred_element_type=jnp.float32)
        return dinv * (msg + t) + b

    h = jax.nn.relu(layer(X, w1_ref[...], b1_ref[...]))
    h = layer(h, w2_ref[...], b2_ref[...])
    out_ref[0] = jnp.mean(h, axis=1)


def kernel(x, adj, W1, b1, W2, b2):
    B, N, F = x.shape
    O = W2.shape[1]
    g = _PER_PROG
    return pl.pallas_call(
        _gcn_body,
        grid=(B // g,),
        in_specs=[
            pl.BlockSpec((g, N, F), lambda b: (b, 0, 0)),
            pl.BlockSpec((g, N, N), lambda b: (b, 0, 0)),
            pl.BlockSpec(W1.shape, lambda b: (0, 0)),
            pl.BlockSpec((1, b1.shape[0]), lambda b: (0, 0)),
            pl.BlockSpec(W2.shape, lambda b: (0, 0)),
            pl.BlockSpec((1, b2.shape[0]), lambda b: (0, 0)),
        ],
        out_specs=pl.BlockSpec((1, g, O), lambda b: (b, 0, 0)),
        out_shape=jax.ShapeDtypeStruct((B // g, g, O), x.dtype),
    )(x, adj, W1, b1.reshape(1, -1), W2, b2.reshape(1, -1)).reshape(B, O)
